# baseline (speedup 1.0000x reference)
import functools

import jax
import jax.numpy as jnp
import numpy as np
from jax.experimental import pallas as pl

K = 32
BLOCK = 32768


def _fused_block(xt_ref, zt_ref, w1t_ref, w2t_ref, w3t_ref,
                 tx_ref, dtx_ref):
    f32 = jnp.float32
    hp = jax.lax.Precision.DEFAULT
    zt = zt_ref[...]
    ones_r = jnp.ones((2, zt.shape[1]), f32)
    zaug = jnp.concatenate([zt, ones_r], axis=0)
    h = jnp.maximum(jnp.dot(w1t_ref[...], zaug,
                            preferred_element_type=f32), 0.0)
    h = jnp.maximum(jnp.dot(w2t_ref[...], h,
                            preferred_element_type=f32), 0.0)
    theta = jnp.dot(w3t_ref[...], h, preferred_element_type=f32)
    tw = theta[0:K, :]
    th = theta[K:2 * K, :]
    td = theta[2 * K:3 * K, :]

    ew = jnp.exp(tw - jnp.max(tw, axis=0, keepdims=True))
    eh = jnp.exp(th - jnp.max(th, axis=0, keepdims=True))
    r = jax.lax.broadcasted_iota(jnp.int32, (K, K), 0)
    c = jax.lax.broadcasted_iota(jnp.int32, (K, K), 1)
    tri = (r >= c).astype(f32)
    cw = jnp.dot(tri, ew, preferred_element_type=f32, precision=hp)

    xv = xt_ref[...]
    thr = (xv + 1.0) * (cw[K - 1:K, :] * 0.5)
    m = (cw < thr).astype(f32)
    one_row = jnp.ones(xv.shape, f32)
    zero_row = jnp.zeros(xv.shape, f32)
    msd = jnp.concatenate([one_row, m[:K - 1, :]], axis=0)
    msu = jnp.concatenate([m[1:, :], zero_row], axis=0)
    oh = msd - m
    ohm1 = m - msu
    is0 = m[0:1, :] == 0.0

    bsz = ew.shape[1]
    _part = lambda p: jnp.reshape(p, (K // 8, 8, bsz)).sum(axis=0)
    _hi = lambda p: (p.astype(jnp.bfloat16)).astype(f32)
    pexact = [_part(ew), _part(eh), _part(m * ew), _part(m * eh),
              _part(oh * ew), _part(oh * eh)]
    papprox = [_part(oh * td), _part(ohm1 * td)]
    phi = [_hi(p) for p in pexact]
    plo = [p - h for p, h in zip(pexact, phi)]
    parts = jnp.concatenate(phi + plo + papprox, axis=0)
    gi = jax.lax.broadcasted_iota(jnp.int32, (8, 112), 0)
    gj = jax.lax.broadcasted_iota(jnp.int32, (8, 112), 1)
    gjm = gj // 8
    sel = jnp.logical_or(
        jnp.logical_and(gjm == gi + 6, gi < 6),
        jnp.logical_or(jnp.logical_and(gjm == gi, gi < 6),
                       jnp.logical_and(gjm == gi + 6, gi >= 6))
        ).astype(f32)
    red = jnp.dot(sel, parts, preferred_element_type=f32)
    sw = 2.0 / red[0:1, :]
    sh = 2.0 / red[1:2, :]
    x_k = red[2:3, :] * sw - 1.0
    y_k = red[3:4, :] * sh - 1.0
    x_nk = x_k + red[4:5, :] * sw
    y_nk = y_k + red[5:6, :] * sh
    d_nk = jax.nn.softplus(red[6:7, :]) + 0.001
    d_k = jnp.where(is0, 1.0, jax.nn.softplus(red[7:8, :]) + 0.001)

    s_k = (y_nk - y_k) / (x_nk - x_k)
    eps = (xv - x_k) / (x_nk - x_k)
    om = 1.0 - eps
    denom = s_k + (d_nk + d_k - 2.0 * s_k) * eps * om
    tx = y_k + (y_nk - y_k) * (s_k * eps * eps + d_k * eps * om) / denom
    dtx = (s_k * s_k * (d_nk * eps * eps + 2.0 * s_k * eps * om
                        + d_k * om * om) / (denom * denom))
    tx_ref[...] = tx
    dtx_ref[...] = dtx


@jax.jit
def kernel(x, z, W1, b1, W2, b2, W3, b3):
    n = x.shape[0]
    f32 = jnp.float32
    pad_bias = np.log(np.expm1(0.999)).astype(np.float32)
    w3aug = jnp.concatenate([W3, jnp.zeros((W3.shape[0], 1), f32)], axis=1)
    b3aug = jnp.concatenate([b3, jnp.full((1,), pad_bias, f32)])
    def _bsplit(b):
        hi = b.astype(jnp.bfloat16).astype(f32)
        return hi, b - hi
    b1h, b1l = _bsplit(b1)
    b2h, b2l = _bsplit(b2)
    b3h, b3l = _bsplit(b3aug)
    w1t = jnp.zeros((72, 10), f32).at[0:64, 0:8].set(W1.T)
    w1t = w1t.at[0:64, 8].set(b1h).at[0:64, 9].set(b1l)
    w1t = w1t.at[64, 8].set(1.0).at[65, 8].set(1.0)
    w2t = jnp.zeros((72, 72), f32).at[0:64, 0:64].set(W2.T)
    w2t = w2t.at[0:64, 64].set(b2h).at[0:64, 65].set(b2l)
    w2t = w2t.at[64, 64].set(1.0).at[65, 64].set(1.0)
    w3t = jnp.zeros((96, 72), f32).at[:, 0:64].set(w3aug.T)
    w3t = w3t.at[:, 64].set(b3h).at[:, 65].set(b3l)
    zt = z.T
    xt = x.reshape(1, n)

    grid = (n // BLOCK,)
    lane_spec = lambda r: pl.BlockSpec((r, BLOCK), lambda i: (0, i))
    full = lambda a: pl.BlockSpec(a.shape, lambda i: (0,) * a.ndim)
    tx, dtx = pl.pallas_call(
        _fused_block,
        grid=grid,
        in_specs=[
            lane_spec(1), lane_spec(z.shape[1]),
            full(w1t), full(w2t), full(w3t),
        ],
        out_specs=[lane_spec(1), lane_spec(1)],
        out_shape=[
            jax.ShapeDtypeStruct((1, n), f32),
            jax.ShapeDtypeStruct((1, n), f32),
        ],
    )(xt, zt, w1t, w2t, w3t)
    return tx.reshape(n, 1), dtx.reshape(n)


# speedup vs baseline: 1.0647x; 1.0647x over previous
import functools

import jax
import jax.numpy as jnp
import numpy as np
from jax.experimental import pallas as pl

K = 32
BLOCK = 32768


def _fused_block(xt_ref, zt_ref, w1t_ref, w2t_ref, w3t_ref,
                 tx_ref, dtx_ref):
    f32 = jnp.float32
    hp = jax.lax.Precision.DEFAULT
    zt = zt_ref[...]
    ones_r = jnp.ones((1, zt.shape[1]), f32)
    zaug = jnp.concatenate([zt, ones_r], axis=0)
    h = jnp.maximum(jnp.dot(w1t_ref[...], zaug,
                            preferred_element_type=f32), 0.0)
    h = jnp.maximum(jnp.dot(w2t_ref[...], h,
                            preferred_element_type=f32), 0.0)
    theta = jnp.dot(w3t_ref[...], h, preferred_element_type=f32)
    tw = theta[0:K, :]
    th = theta[K:2 * K, :]
    td = theta[2 * K:3 * K, :]

    ew = jnp.exp(tw - jnp.max(tw, axis=0, keepdims=True))
    eh = jnp.exp(th - jnp.max(th, axis=0, keepdims=True))
    r = jax.lax.broadcasted_iota(jnp.int32, (K, K), 0)
    c = jax.lax.broadcasted_iota(jnp.int32, (K, K), 1)
    tri = (r >= c).astype(f32)
    cw = jnp.dot(tri, ew, preferred_element_type=f32, precision=hp)

    xv = xt_ref[...]
    thr = (xv + 1.0) * (cw[K - 1:K, :] * 0.5)
    m = (cw < thr).astype(f32)
    one_row = jnp.ones(xv.shape, f32)
    zero_row = jnp.zeros(xv.shape, f32)
    msd = jnp.concatenate([one_row, m[:K - 1, :]], axis=0)
    msu = jnp.concatenate([m[1:, :], zero_row], axis=0)
    oh = msd - m
    ohm1 = m - msu
    is0 = m[0:1, :] == 0.0

    bsz = ew.shape[1]
    _part = lambda p: jnp.reshape(p, (K // 8, 8, bsz)).sum(axis=0)
    _hi = lambda p: (p.astype(jnp.bfloat16)).astype(f32)
    pexact = [_part(ew), _part(eh), _part(m * ew), _part(m * eh),
              _part(oh * ew), _part(oh * eh)]
    papprox = [_part(oh * td), _part(ohm1 * td)]
    phi = [_hi(p) for p in pexact]
    plo = [p - h for p, h in zip(pexact, phi)]
    parts = jnp.concatenate(phi + plo + papprox, axis=0)
    gi = jax.lax.broadcasted_iota(jnp.int32, (8, 112), 0)
    gj = jax.lax.broadcasted_iota(jnp.int32, (8, 112), 1)
    gjm = gj // 8
    sel = jnp.logical_or(
        jnp.logical_and(gjm == gi + 6, gi < 6),
        jnp.logical_or(jnp.logical_and(gjm == gi, gi < 6),
                       jnp.logical_and(gjm == gi + 6, gi >= 6))
        ).astype(f32)
    red = jnp.dot(sel, parts, preferred_element_type=f32)
    sw = 2.0 / red[0:1, :]
    sh = 2.0 / red[1:2, :]
    x_k = red[2:3, :] * sw - 1.0
    y_k = red[3:4, :] * sh - 1.0
    x_nk = x_k + red[4:5, :] * sw
    y_nk = y_k + red[5:6, :] * sh
    d_nk = jax.nn.softplus(red[6:7, :]) + 0.001
    d_k = jnp.where(is0, 1.0, jax.nn.softplus(red[7:8, :]) + 0.001)

    s_k = (y_nk - y_k) / (x_nk - x_k)
    eps = (xv - x_k) / (x_nk - x_k)
    om = 1.0 - eps
    denom = s_k + (d_nk + d_k - 2.0 * s_k) * eps * om
    tx = y_k + (y_nk - y_k) * (s_k * eps * eps + d_k * eps * om) / denom
    dtx = (s_k * s_k * (d_nk * eps * eps + 2.0 * s_k * eps * om
                        + d_k * om * om) / (denom * denom))
    tx_ref[...] = tx
    dtx_ref[...] = dtx


@jax.jit
def kernel(x, z, W1, b1, W2, b2, W3, b3):
    n = x.shape[0]
    f32 = jnp.float32
    pad_bias = np.log(np.expm1(0.999)).astype(np.float32)
    w3aug = jnp.concatenate([W3, jnp.zeros((W3.shape[0], 1), f32)], axis=1)
    b3aug = jnp.concatenate([b3, jnp.full((1,), pad_bias, f32)])
    w1t = jnp.zeros((72, 9), f32).at[0:64, 0:8].set(W1.T)
    w1t = w1t.at[0:64, 8].set(b1).at[64, 8].set(1.0)
    w2t = jnp.zeros((72, 72), f32).at[0:64, 0:64].set(W2.T)
    w2t = w2t.at[0:64, 64].set(b2).at[64, 64].set(1.0)
    w3t = jnp.zeros((96, 72), f32).at[:, 0:64].set(w3aug.T)
    w3t = w3t.at[:, 64].set(b3aug)
    zt = z.T
    xt = x.reshape(1, n)

    grid = (n // BLOCK,)
    lane_spec = lambda r: pl.BlockSpec((r, BLOCK), lambda i: (0, i))
    full = lambda a: pl.BlockSpec(a.shape, lambda i: (0,) * a.ndim)
    tx, dtx = pl.pallas_call(
        _fused_block,
        grid=grid,
        in_specs=[
            lane_spec(1), lane_spec(z.shape[1]),
            full(w1t), full(w2t), full(w3t),
        ],
        out_specs=[lane_spec(1), lane_spec(1)],
        out_shape=[
            jax.ShapeDtypeStruct((1, n), f32),
            jax.ShapeDtypeStruct((1, n), f32),
        ],
    )(xt, zt, w1t, w2t, w3t)
    return tx.reshape(n, 1), dtx.reshape(n)


# speedup vs baseline: 1.0651x; 1.0004x over previous
import jax
import jax.numpy as jnp
import numpy as np
from jax.experimental import pallas as pl

K = 32
BLOCK = 32768


def _fused_block(xt_ref, zt_ref, w1t_ref, w2t_ref, w3t_ref,
                 tx_ref, dtx_ref):
    f32 = jnp.float32
    hp = jax.lax.Precision.DEFAULT
    zt = zt_ref[...]
    ones_r = jnp.ones((1, zt.shape[1]), f32)
    zaug = jnp.concatenate([zt, ones_r], axis=0)
    h = jnp.maximum(jnp.dot(w1t_ref[...], zaug,
                            preferred_element_type=f32), 0.0)
    h = jnp.maximum(jnp.dot(w2t_ref[...], h,
                            preferred_element_type=f32), 0.0)
    theta = jnp.dot(w3t_ref[...], h, preferred_element_type=f32)
    tw = theta[0:K, :]
    th = theta[K:2 * K, :]
    td = theta[2 * K:3 * K, :]

    ew = jnp.exp(tw - jnp.max(tw, axis=0, keepdims=True))
    eh = jnp.exp(th - jnp.max(th, axis=0, keepdims=True))
    r = jax.lax.broadcasted_iota(jnp.int32, (K, K), 0)
    c = jax.lax.broadcasted_iota(jnp.int32, (K, K), 1)
    tri = (r >= c).astype(f32)
    cw = jnp.dot(tri, ew, preferred_element_type=f32, precision=hp)

    xv = xt_ref[...]
    thr = (xv + 1.0) * (cw[K - 1:K, :] * 0.5)
    m = (cw < thr).astype(f32)
    one_row = jnp.ones(xv.shape, f32)
    zero_row = jnp.zeros(xv.shape, f32)
    msd = jnp.concatenate([one_row, m[:K - 1, :]], axis=0)
    msu = jnp.concatenate([m[1:, :], zero_row], axis=0)
    oh = msd - m
    ohm1 = m - msu
    is0 = m[0:1, :] == 0.0

    bsz = ew.shape[1]
    _part = lambda p: jnp.reshape(p, (K // 8, 8, bsz)).sum(axis=0)
    _hi = lambda p: (p.astype(jnp.bfloat16)).astype(f32)
    pexact = [_part(ew), _part(eh), _part(m * ew), _part(m * eh),
              _part(oh * ew), _part(oh * eh)]
    papprox = [_part(oh * td), _part(ohm1 * td)]
    phi = [_hi(p) for p in pexact]
    plo = [p - h for p, h in zip(pexact, phi)]
    parts = jnp.concatenate(phi + plo + papprox, axis=0)
    gi = jax.lax.broadcasted_iota(jnp.int32, (8, 112), 0)
    gj = jax.lax.broadcasted_iota(jnp.int32, (8, 112), 1)
    gjm = gj // 8
    sel = jnp.logical_or(
        jnp.logical_and(gjm == gi + 6, gi < 6),
        jnp.logical_or(jnp.logical_and(gjm == gi, gi < 6),
                       jnp.logical_and(gjm == gi + 6, gi >= 6))
        ).astype(f32)
    red = jnp.dot(sel, parts, preferred_element_type=f32)
    sw = 2.0 / red[0:1, :]
    sh = 2.0 / red[1:2, :]
    x_k = red[2:3, :] * sw - 1.0
    y_k = red[3:4, :] * sh - 1.0
    x_nk = x_k + red[4:5, :] * sw
    y_nk = y_k + red[5:6, :] * sh
    d_nk = jax.nn.softplus(red[6:7, :]) + 0.001
    d_k = jnp.where(is0, 1.0, jax.nn.softplus(red[7:8, :]) + 0.001)

    s_k = (y_nk - y_k) / (x_nk - x_k)
    eps = (xv - x_k) / (x_nk - x_k)
    om = 1.0 - eps
    denom = s_k + (d_nk + d_k - 2.0 * s_k) * eps * om
    tx = y_k + (y_nk - y_k) * (s_k * eps * eps + d_k * eps * om) / denom
    dtx = (s_k * s_k * (d_nk * eps * eps + 2.0 * s_k * eps * om
                        + d_k * om * om) / (denom * denom))
    tx_ref[...] = tx
    dtx_ref[...] = dtx


@jax.jit
def kernel(x, z, W1, b1, W2, b2, W3, b3):
    n = x.shape[0]
    f32 = jnp.float32
    pad_bias = np.log(np.expm1(0.999)).astype(np.float32)
    w3aug = jnp.concatenate([W3, jnp.zeros((W3.shape[0], 1), f32)], axis=1)
    b3aug = jnp.concatenate([b3, jnp.full((1,), pad_bias, f32)])
    w1t = jnp.zeros((72, 9), f32).at[0:64, 0:8].set(W1.T)
    w1t = w1t.at[0:64, 8].set(b1).at[64, 8].set(1.0)
    w2t = jnp.zeros((72, 72), f32).at[0:64, 0:64].set(W2.T)
    w2t = w2t.at[0:64, 64].set(b2).at[64, 64].set(1.0)
    w3t = jnp.zeros((96, 72), f32).at[:, 0:64].set(w3aug.T)
    w3t = w3t.at[:, 64].set(b3aug)
    zt = z.T
    xt = x.reshape(1, n)

    grid = (n // BLOCK,)
    lane_spec = lambda r: pl.BlockSpec((r, BLOCK), lambda i: (0, i))
    full = lambda a: pl.BlockSpec(a.shape, lambda i: (0,) * a.ndim)
    tx, dtx = pl.pallas_call(
        _fused_block,
        grid=grid,
        in_specs=[
            lane_spec(1), lane_spec(z.shape[1]),
            full(w1t), full(w2t), full(w3t),
        ],
        out_specs=[lane_spec(1), lane_spec(1)],
        out_shape=[
            jax.ShapeDtypeStruct((1, n), f32),
            jax.ShapeDtypeStruct((1, n), f32),
        ],
    )(xt, zt, w1t, w2t, w3t)
    return tx.reshape(n, 1), dtx.reshape(n)
